# baseline (speedup 1.0000x reference)
import functools

import jax
import jax.numpy as jnp
from jax import lax
from jax.experimental import pallas as pl
from jax.experimental.pallas import tpu as pltpu
from jax.experimental.pallas import tpu_sc as plsc

_ITEM = 1000000
_H = 64
_L = 200
_B = 4096
_TOTAL = _B * _L
_NW = 32
_PER_W = _TOTAL // _NW
_CHUNK = 512
_NCH = _PER_W // _CHUNK
_ISUB = 128
_NSUB = _CHUNK // _ISUB
_NLANE = 16


def _rsqrt_newton(x):
    i = lax.bitcast_convert_type(x, jnp.int32)
    i = jnp.int32(0x5F3759DF) - (i >> 1)
    y = lax.bitcast_convert_type(i, jnp.float32)
    for _ in range(4):
        y = y * (1.5 - 0.5 * x * y * y)
    return y


def _sc_body(table_h, ids_h, pos_h, gamma_h, beta_h, out_h,
             idx_v, rows_v, pos_v, gb_v, sem):
    wid = lax.axis_index("s") * 2 + lax.axis_index("c")

    pltpu.sync_copy(pos_h, pos_v)
    pltpu.sync_copy(gamma_h, gb_v.at[0])
    pltpu.sync_copy(beta_h, gb_v.at[1])

    g_regs = [gb_v[0, pl.ds(k * _NLANE, _NLANE)] for k in range(_H // _NLANE)]
    b_regs = [gb_v[1, pl.ds(k * _NLANE, _NLANE)] for k in range(_H // _NLANE)]

    lane = lax.iota(jnp.int32, _NLANE)
    perms = [(lane ^ sh).reshape(_NLANE, 1) for sh in (8, 4, 2, 1)]
    _dnums = lax.GatherDimensionNumbers(
        offset_dims=(), collapsed_slice_dims=(0,), start_index_map=(0,))

    def allsum(v):
        for p in perms:
            v = v + lax.gather(
                v, p, dimension_numbers=_dnums, slice_sizes=(1,),
                mode=lax.GatherScatterMode.PROMISE_IN_BOUNDS)
        return v

    def chunk_body(c, carry):
        base = wid * _PER_W + c * _CHUNK
        idrow = wid * (_PER_W // _ISUB) + c * _NSUB
        pltpu.sync_copy(ids_h.at[pl.ds(idrow, _NSUB)], idx_v)
        cps = [
            pltpu.async_copy(
                table_h.at[idx_v.at[j]],
                rows_v.at[pl.ds(j * _ISUB, _ISUB)],
                sem,
            )
            for j in range(_NSUB)
        ]
        for cp in cps:
            cp.wait()

        def row_body(i, carry2):
            lpos = (base + i) % _L
            xs = []
            for k in range(_H // _NLANE):
                v = rows_v[i, pl.ds(k * _NLANE, _NLANE)]
                p = pos_v[lpos, pl.ds(k * _NLANE, _NLANE)]
                xs.append(v + p)
            t = (xs[0] + xs[1]) + (xs[2] + xs[3])
            q = (xs[0] * xs[0] + xs[1] * xs[1]) + (xs[2] * xs[2] + xs[3] * xs[3])
            s_v = allsum(t)
            q_v = allsum(q)
            mu = s_v * (1.0 / _H)
            var = q_v * (1.0 / _H) - mu * mu
            rstd = _rsqrt_newton(var + 1e-12)
            for k in range(_H // _NLANE):
                o = (xs[k] - mu) * rstd * g_regs[k] + b_regs[k]
                rows_v[i, pl.ds(k * _NLANE, _NLANE)] = o
            return carry2

        lax.fori_loop(0, _CHUNK, row_body, 0, unroll=False)
        pltpu.sync_copy(rows_v, out_h.at[pl.ds(base, _CHUNK)])
        return carry

    lax.fori_loop(0, _NCH, chunk_body, 0, unroll=False)


def kernel(input_ids, item_table, pos_table, ln_gamma, ln_beta):
    ids2d = input_ids.reshape(_TOTAL // _ISUB, _ISUB)
    mesh = plsc.VectorSubcoreMesh(core_axis_name="c", subcore_axis_name="s")
    run = functools.partial(
        pl.kernel,
        mesh=mesh,
        compiler_params=pltpu.CompilerParams(use_tc_tiling_on_sc=False),
        out_type=jax.ShapeDtypeStruct((_TOTAL, _H), jnp.float32),
        scratch_types=[
            pltpu.VMEM((_NSUB, _ISUB), jnp.int32),
            pltpu.VMEM((_CHUNK, _H), jnp.float32),
            pltpu.VMEM((_L, _H), jnp.float32),
            pltpu.VMEM((2, _H), jnp.float32),
            pltpu.SemaphoreType.DMA,
        ],
    )(_sc_body)
    out = run(item_table, ids2d, pos_table, ln_gamma, ln_beta)
    return out.reshape(_B, _L, _H)


# speedup vs baseline: 1.1789x; 1.1789x over previous
import functools

import jax
import jax.numpy as jnp
from jax import lax
from jax.experimental import pallas as pl
from jax.experimental.pallas import tpu as pltpu
from jax.experimental.pallas import tpu_sc as plsc

_ITEM = 1000000
_H = 64
_L = 200
_B = 4096
_TOTAL = _B * _L
_NW = 32
_PER_W = _TOTAL // _NW
_CHUNK = 256
_NCH = _PER_W // _CHUNK
_ISUB = 128
_NSUB = _CHUNK // _ISUB
_IDR_W = _PER_W // _ISUB
_NLANE = 16
_NBUF = 2


def _rsqrt_newton(x):
    i = lax.bitcast_convert_type(x, jnp.int32)
    i = jnp.int32(0x5F3759DF) - (i >> 1)
    y = lax.bitcast_convert_type(i, jnp.float32)
    for _ in range(2):
        y = y * (1.5 - 0.5 * x * y * y)
    return y


def _sc_body(table_h, ids_h, pos_h, gamma_h, beta_h, out_h,
             ids_v, in_v, out_v, pos_v, gb_v, sg0, sg1, sw0, sw1):
    wid = lax.axis_index("s") * 2 + lax.axis_index("c")
    sg = [sg0, sg1]
    sw = [sw0, sw1]

    pltpu.sync_copy(ids_h.at[pl.ds(wid * _IDR_W, _IDR_W)], ids_v)
    pltpu.sync_copy(pos_h, pos_v)
    pltpu.sync_copy(gamma_h, gb_v.at[0])
    pltpu.sync_copy(beta_h, gb_v.at[1])

    g_regs = [gb_v[0, pl.ds(k * _NLANE, _NLANE)] for k in range(_H // _NLANE)]
    b_regs = [gb_v[1, pl.ds(k * _NLANE, _NLANE)] for k in range(_H // _NLANE)]

    lane = lax.iota(jnp.int32, _NLANE)
    perms = [(lane ^ sh).reshape(_NLANE, 1) for sh in (8, 4, 2, 1)]
    _dnums = lax.GatherDimensionNumbers(
        offset_dims=(), collapsed_slice_dims=(0,), start_index_map=(0,))

    def allsum(v):
        for p in perms:
            v = v + lax.gather(
                v, p, dimension_numbers=_dnums, slice_sizes=(1,),
                mode=lax.GatherScatterMode.PROMISE_IN_BOUNDS)
        return v

    def start_gather(c, b):
        for j in range(_NSUB):
            pltpu.async_copy(
                table_h.at[ids_v.at[c * _NSUB + j]],
                in_v.at[b].at[pl.ds(j * _ISUB, _ISUB)],
                sg[b])

    def wait_gather(c, b):
        for j in range(_NSUB):
            pltpu.make_async_copy(
                table_h.at[ids_v.at[c * _NSUB + j]],
                in_v.at[b].at[pl.ds(j * _ISUB, _ISUB)],
                sg[b]).wait()

    def start_write(c, b):
        pltpu.async_copy(
            out_v.at[b], out_h.at[pl.ds(wid * _PER_W + c * _CHUNK, _CHUNK)],
            sw[b])

    def wait_write(c, b):
        pltpu.make_async_copy(
            out_v.at[b], out_h.at[pl.ds(wid * _PER_W + c * _CHUNK, _CHUNK)],
            sw[b]).wait()

    def compute(c, b):
        lp0 = (c * _CHUNK) % _L

        def row_body(i, carry2):
            lpos = lp0 + i
            lpos = jnp.where(lpos >= 2 * _L, lpos - 2 * _L,
                             jnp.where(lpos >= _L, lpos - _L, lpos))
            xs = []
            for k in range(_H // _NLANE):
                v = in_v[b, i, pl.ds(k * _NLANE, _NLANE)]
                p = pos_v[lpos, pl.ds(k * _NLANE, _NLANE)]
                xs.append(v + p)
            t = (xs[0] + xs[1]) + (xs[2] + xs[3])
            q = (xs[0] * xs[0] + xs[1] * xs[1]) + (xs[2] * xs[2] + xs[3] * xs[3])
            s_v = allsum(t)
            q_v = allsum(q)
            mu = s_v * (1.0 / _H)
            var = q_v * (1.0 / _H) - mu * mu
            rstd = _rsqrt_newton(var + 1e-12)
            for k in range(_H // _NLANE):
                o = (xs[k] - mu) * rstd * g_regs[k] + b_regs[k]
                out_v[b, i, pl.ds(k * _NLANE, _NLANE)] = o
            return carry2

        lax.fori_loop(0, _CHUNK, row_body, 0, unroll=4)

    for b in range(_NBUF):
        start_gather(b, b)

    for b in range(_NBUF):
        wait_gather(b, b)
        compute(b, b)
        start_write(b, b)
        start_gather(b + _NBUF, b)

    def group_body(g, carry):
        for b in range(_NBUF):
            c = g * _NBUF + b
            wait_gather(c, b)
            wait_write(c - _NBUF, b)
            compute(c, b)
            start_write(c, b)
            start_gather(c + _NBUF, b)
        return carry

    lax.fori_loop(1, _NCH // _NBUF - 1, group_body, 0, unroll=False)

    for b in range(_NBUF):
        c = _NCH - _NBUF + b
        wait_gather(c, b)
        wait_write(c - _NBUF, b)
        compute(c, b)
        start_write(c, b)
    for b in range(_NBUF):
        wait_write(_NCH - _NBUF + b, b)


def kernel(input_ids, item_table, pos_table, ln_gamma, ln_beta):
    ids2d = input_ids.reshape(_TOTAL // _ISUB, _ISUB)
    mesh = plsc.VectorSubcoreMesh(core_axis_name="c", subcore_axis_name="s")
    run = functools.partial(
        pl.kernel,
        mesh=mesh,
        compiler_params=pltpu.CompilerParams(use_tc_tiling_on_sc=False),
        out_type=jax.ShapeDtypeStruct((_TOTAL, _H), jnp.float32),
        scratch_types=[
            pltpu.VMEM((_IDR_W, _ISUB), jnp.int32),
            pltpu.VMEM((_NBUF, _CHUNK, _H), jnp.float32),
            pltpu.VMEM((_NBUF, _CHUNK, _H), jnp.float32),
            pltpu.VMEM((_L, _H), jnp.float32),
            pltpu.VMEM((2, _H), jnp.float32),
            pltpu.SemaphoreType.DMA,
            pltpu.SemaphoreType.DMA,
            pltpu.SemaphoreType.DMA,
            pltpu.SemaphoreType.DMA,
        ],
    )(_sc_body)
    out = run(item_table, ids2d, pos_table, ln_gamma, ln_beta)
    return out.reshape(_B, _L, _H)
